# baseline (speedup 1.0000x reference)
import functools

import jax
import jax.numpy as jnp
from jax import lax
from jax.experimental import pallas as pl
from jax.experimental.pallas import tpu as pltpu
from jax.experimental.pallas import tpu_sc as plsc

B = 4 * 8192
D = 1024
V = 8192
NC, NS = 2, 16
NW = NC * NS
B_PER_W = B // NW
CHUNK = 32
NCHUNK = B_PER_W // CHUNK


def _gather_kernel(table_hbm, idx_hbm, out_hbm, idx_v, buf0, buf1, sem0, sem1):
    wid = lax.axis_index("s") * NC + lax.axis_index("c")
    base = wid * B_PER_W
    pltpu.sync_copy(idx_hbm.at[pl.ds(base, B_PER_W)], idx_v)

    def gather_cp(g, buf, sem):
        return pltpu.make_async_copy(
            table_hbm.at[idx_v.at[pl.ds(g * CHUNK, CHUNK)]], buf, sem
        )

    def write(g, buf):
        pltpu.sync_copy(buf, out_hbm.at[pl.ds(base + g * CHUNK, CHUNK)])

    gather_cp(0, buf0, sem0).start()

    @pl.loop(0, NCHUNK, step=2)
    def _(g):
        gather_cp(g + 1, buf1, sem1).start()
        gather_cp(g, buf0, sem0).wait()
        write(g, buf0)

        @pl.when(g + 2 < NCHUNK)
        def _():
            gather_cp(g + 2, buf0, sem0).start()

        gather_cp(g + 1, buf1, sem1).wait()
        write(g + 1, buf1)


def kernel(position_ids, embedding_weight):
    idx = position_ids.reshape(B).astype(jnp.int32)
    table3 = embedding_weight.reshape(V, 8, 128)
    mesh = plsc.VectorSubcoreMesh(core_axis_name="c", subcore_axis_name="s")
    k = functools.partial(
        pl.kernel,
        mesh=mesh,
        out_type=jax.ShapeDtypeStruct((B, 8, 128), jnp.float32),
        scratch_types=[
            pltpu.VMEM((B_PER_W,), jnp.int32),
            pltpu.VMEM((CHUNK, 8, 128), jnp.float32),
            pltpu.VMEM((CHUNK, 8, 128), jnp.float32),
            pltpu.SemaphoreType.DMA,
            pltpu.SemaphoreType.DMA,
        ],
    )(_gather_kernel)
    out = k(table3, idx)
    return out.reshape(4, 8192, D)


# speedup vs baseline: 2.4287x; 2.4287x over previous
import functools

import jax
import jax.numpy as jnp
from jax import lax
from jax.experimental import pallas as pl
from jax.experimental.pallas import tpu as pltpu
from jax.experimental.pallas import tpu_sc as plsc

B = 4 * 8192
D = 1024
NC, NS = 2, 16
NW = NC * NS
B_PER_W = B // NW
CHUNK = 32
NCHUNK = B_PER_W // CHUNK


def _gather_kernel(table_hbm, idx_hbm, out_hbm, idx_v, buf0, buf1, sem0, sem1):
    wid = lax.axis_index("s") * NC + lax.axis_index("c")
    base = wid * B_PER_W
    pltpu.sync_copy(idx_hbm.at[pl.ds(base, B_PER_W)], idx_v)

    def gather_cp(g, buf, sem):
        return pltpu.make_async_copy(
            table_hbm.at[idx_v.at[pl.ds(g * CHUNK, CHUNK)]], buf, sem
        )

    def write(g, buf):
        pltpu.sync_copy(buf, out_hbm.at[pl.ds(base + g * CHUNK, CHUNK)])

    gather_cp(0, buf0, sem0).start()

    @pl.loop(0, NCHUNK, step=2)
    def _(g):
        gather_cp(g + 1, buf1, sem1).start()
        gather_cp(g, buf0, sem0).wait()
        write(g, buf0)

        @pl.when(g + 2 < NCHUNK)
        def _():
            gather_cp(g + 2, buf0, sem0).start()

        gather_cp(g + 1, buf1, sem1).wait()
        write(g + 1, buf1)


def kernel(position_ids, embedding_weight):
    idx = position_ids.reshape(B).astype(jnp.int32)
    mesh = plsc.VectorSubcoreMesh(core_axis_name="c", subcore_axis_name="s")
    k = functools.partial(
        pl.kernel,
        mesh=mesh,
        out_type=jax.ShapeDtypeStruct((B, D), jnp.float32),
        scratch_types=[
            pltpu.VMEM((B_PER_W,), jnp.int32),
            pltpu.VMEM((CHUNK, D), jnp.float32),
            pltpu.VMEM((CHUNK, D), jnp.float32),
            pltpu.SemaphoreType.DMA,
            pltpu.SemaphoreType.DMA,
        ],
    )(_gather_kernel)
    out = k(embedding_weight, idx)
    return out.reshape(4, 8192, D)
